# Initial kernel scaffold; baseline (speedup 1.0000x reference)
import jax
import jax.numpy as jnp
from jax.experimental import pallas as pl


def kernel(preds, targets):
    raise NotImplementedError("write your pallas kernel here")


# speedup vs baseline: 87.7909x; 87.7909x over previous
import jax
import jax.numpy as jnp
from jax.experimental import pallas as pl
from jax.experimental.pallas import tpu as pltpu

_ALPHA = 0.5
_B, _H, _W, _D = 2, 48, 48, 48
_L = _W * _D
_N = _H * _W * _D
_BIG = jnp.iinfo(jnp.int32).max


def _stencil(x, op, fill):
    shp = x.shape
    f = lambda s: jnp.full(s, fill, x.dtype)
    lo = jnp.concatenate([f(shp[:-2] + (1, shp[-1])), x[..., :-1, :]], axis=-2)
    hi = jnp.concatenate([x[..., 1:, :], f(shp[:-2] + (1, shp[-1]))], axis=-2)
    x = op(op(x, lo), hi)
    lo = jnp.concatenate([f(shp[:-1] + (_D,)), x[..., :-_D]], axis=-1)
    hi = jnp.concatenate([x[..., _D:], f(shp[:-1] + (_D,))], axis=-1)
    x = op(op(x, lo), hi)
    col = jax.lax.broadcasted_iota(jnp.int32, shp, len(shp) - 1)
    cm = col % _D
    lo = jnp.concatenate([f(shp[:-1] + (1,)), x[..., :-1]], axis=-1)
    lo = jnp.where(cm == 0, fill, lo)
    hi = jnp.concatenate([x[..., 1:], f(shp[:-1] + (1,))], axis=-1)
    hi = jnp.where(cm == _D - 1, fill, hi)
    x = op(op(x, lo), hi)
    return x


def _loss_kernel(p0_ref, p1_ref, t_ref, out_ref, lab_ref, m_ref, mv_ref, mm_ref):
    t1 = t_ref[:] == 1
    psq = p1_ref[:] > p0_ref[:]
    agree = jnp.logical_and(t1, psq)

    row = jax.lax.broadcasted_iota(jnp.int32, (_B, _H, _L), 1)
    col = jax.lax.broadcasted_iota(jnp.int32, (_B, _H, _L), 2)
    idx = row * _L + col + 1
    lab_ref[:] = jnp.where(agree, idx, 0)
    m_ref[:] = agree.astype(jnp.int32)

    def ccl_body(_):
        lab = lab_ref[:]
        ag = m_ref[:] != 0
        new = jnp.where(ag, _stencil(lab, jnp.maximum, 0), 0)
        lab_ref[:] = new
        return jnp.any(new != lab)

    jax.lax.while_loop(lambda c: c, ccl_body, jnp.bool_(True))

    lab = lab_ref[:]
    agree = m_ref[:] != 0
    nmax = _stencil(jnp.where(agree, lab, 0), jnp.maximum, 0)
    nmin = _stencil(jnp.where(agree, lab, _BIG), jnp.minimum, _BIG)
    nminc = jnp.where(nmin == _BIG, 0, _N + 1 - nmin)

    fn = jnp.logical_and(t1, jnp.logical_not(psq))
    fp = jnp.logical_and(psq, jnp.logical_not(t1))

    mm_ref[:] = jnp.concatenate([fn, fn, fp, fp], axis=0).astype(jnp.int32)
    mv_ref[:] = jnp.concatenate(
        [jnp.where(fn, nminc, 0), jnp.where(fn, nmax, 0),
         jnp.where(fp, nminc, 0), jnp.where(fp, nmax, 0)], axis=0)

    def mist_body(_):
        v = mv_ref[:]
        mk = mm_ref[:] != 0
        new = jnp.where(mk, _stencil(v, jnp.maximum, 0), 0)
        mv_ref[:] = new
        return jnp.any(new != v)

    jax.lax.while_loop(lambda c: c, mist_body, jnp.bool_(True))

    v = mv_ref[:]
    cminc_fn, cmax_fn = v[0:_B], v[_B:2 * _B]
    cminc_fp, cmax_fp = v[2 * _B:3 * _B], v[3 * _B:4 * _B]
    single_fn = jnp.logical_and(cmax_fn > 0, (_N + 1 - cminc_fn) == cmax_fn)
    single_fp = jnp.logical_and(cmax_fp > 0, (_N + 1 - cminc_fp) == cmax_fp)
    crit_fn = jnp.logical_and(fn, jnp.logical_not(single_fn))
    crit_fp = jnp.logical_and(fp, jnp.logical_not(single_fp))
    combined = crit_fn.astype(jnp.float32) + crit_fp.astype(jnp.float32)

    p0, p1 = p0_ref[:], p1_ref[:]
    mx = jnp.maximum(p0, p1)
    lse = mx + jnp.log(jnp.exp(p0 - mx) + jnp.exp(p1 - mx))
    ce = lse - jnp.where(t1, p1, p0)
    total = jnp.sum((1.0 - _ALPHA + combined) * ce) / (_B * _N)
    out_ref[:] = total[None, None]


def kernel(preds, targets):
    p0 = preds[:, 0].reshape(_B, _H, _L)
    p1 = preds[:, 1].reshape(_B, _H, _L)
    t = targets[:, 0].reshape(_B, _H, _L)
    out = pl.pallas_call(
        _loss_kernel,
        out_shape=jax.ShapeDtypeStruct((1, 1), jnp.float32),
        scratch_shapes=[
            pltpu.VMEM((_B, _H, _L), jnp.int32),
            pltpu.VMEM((_B, _H, _L), jnp.int32),
            pltpu.VMEM((4 * _B, _H, _L), jnp.int32),
            pltpu.VMEM((4 * _B, _H, _L), jnp.int32),
        ],
    )(p0, p1, t)
    return out[0, 0]


# speedup vs baseline: 91.6839x; 1.0443x over previous
import jax
import jax.numpy as jnp
from jax.experimental import pallas as pl
from jax.experimental.pallas import tpu as pltpu

_ALPHA = 0.5
_B, _H, _W, _D = 2, 48, 48, 48
_L = _W * _D
_N = _H * _W * _D
_BIG = jnp.iinfo(jnp.int32).max


def _stencil(x, op, fill):
    shp = x.shape
    last = len(shp) - 1
    f = lambda s: jnp.full(s, fill, x.dtype)
    lo = jnp.concatenate([f(shp[:-2] + (1, shp[-1])), x[..., :-1, :]], axis=-2)
    hi = jnp.concatenate([x[..., 1:, :], f(shp[:-2] + (1, shp[-1]))], axis=-2)
    x = op(op(x, lo), hi)
    col = jax.lax.broadcasted_iota(jnp.int32, shp, last)
    lo = jnp.where(col < _D, fill, pltpu.roll(x, _D, last))
    hi = jnp.where(col >= _L - _D, fill, pltpu.roll(x, _L - _D, last))
    x = op(op(x, lo), hi)
    cm = col % _D
    lo = pltpu.roll(jnp.where(cm == _D - 1, fill, x), 1, last)
    hi = pltpu.roll(jnp.where(cm == 0, fill, x), _L - 1, last)
    x = op(op(x, lo), hi)
    return x


def _loss_kernel(p0_ref, p1_ref, t_ref, out_ref, lab_ref, m_ref, mv_ref, mm_ref):
    t1 = t_ref[:] == 1
    psq = p1_ref[:] > p0_ref[:]
    agree = jnp.logical_and(t1, psq)

    row = jax.lax.broadcasted_iota(jnp.int32, (_B, _H, _L), 1)
    col = jax.lax.broadcasted_iota(jnp.int32, (_B, _H, _L), 2)
    idx = row * _L + col + 1
    lab_ref[:] = jnp.where(agree, idx, 0)
    m_ref[:] = agree.astype(jnp.int32)

    def ccl_body(_):
        lab = lab_ref[:]
        ag = m_ref[:] != 0
        new = jnp.where(ag, _stencil(lab, jnp.maximum, 0), 0)
        lab_ref[:] = new
        return jnp.any(new != lab)

    jax.lax.while_loop(lambda c: c, ccl_body, jnp.bool_(True))

    lab = lab_ref[:]
    agree = m_ref[:] != 0
    nmax = _stencil(jnp.where(agree, lab, 0), jnp.maximum, 0)
    nmin = _stencil(jnp.where(agree, lab, _BIG), jnp.minimum, _BIG)
    nminc = jnp.where(nmin == _BIG, 0, _N + 1 - nmin)

    fn = jnp.logical_and(t1, jnp.logical_not(psq))
    fp = jnp.logical_and(psq, jnp.logical_not(t1))

    mm_ref[:] = jnp.concatenate([fn, fn, fp, fp], axis=0).astype(jnp.int32)
    mv_ref[:] = jnp.concatenate(
        [jnp.where(fn, nminc, 0), jnp.where(fn, nmax, 0),
         jnp.where(fp, nminc, 0), jnp.where(fp, nmax, 0)], axis=0)

    def mist_body(_):
        v = mv_ref[:]
        mk = mm_ref[:] != 0
        new = jnp.where(mk, _stencil(v, jnp.maximum, 0), 0)
        mv_ref[:] = new
        return jnp.any(new != v)

    jax.lax.while_loop(lambda c: c, mist_body, jnp.bool_(True))

    v = mv_ref[:]
    cminc_fn, cmax_fn = v[0:_B], v[_B:2 * _B]
    cminc_fp, cmax_fp = v[2 * _B:3 * _B], v[3 * _B:4 * _B]
    single_fn = jnp.logical_and(cmax_fn > 0, (_N + 1 - cminc_fn) == cmax_fn)
    single_fp = jnp.logical_and(cmax_fp > 0, (_N + 1 - cminc_fp) == cmax_fp)
    crit_fn = jnp.logical_and(fn, jnp.logical_not(single_fn))
    crit_fp = jnp.logical_and(fp, jnp.logical_not(single_fp))
    combined = crit_fn.astype(jnp.float32) + crit_fp.astype(jnp.float32)

    p0, p1 = p0_ref[:], p1_ref[:]
    mx = jnp.maximum(p0, p1)
    lse = mx + jnp.log(jnp.exp(p0 - mx) + jnp.exp(p1 - mx))
    ce = lse - jnp.where(t1, p1, p0)
    total = jnp.sum((1.0 - _ALPHA + combined) * ce) / (_B * _N)
    out_ref[:] = total[None, None]


def kernel(preds, targets):
    p0 = preds[:, 0].reshape(_B, _H, _L)
    p1 = preds[:, 1].reshape(_B, _H, _L)
    t = targets[:, 0].reshape(_B, _H, _L)
    out = pl.pallas_call(
        _loss_kernel,
        out_shape=jax.ShapeDtypeStruct((1, 1), jnp.float32),
        scratch_shapes=[
            pltpu.VMEM((_B, _H, _L), jnp.int32),
            pltpu.VMEM((_B, _H, _L), jnp.int32),
            pltpu.VMEM((4 * _B, _H, _L), jnp.int32),
            pltpu.VMEM((4 * _B, _H, _L), jnp.int32),
        ],
    )(p0, p1, t)
    return out[0, 0]
